# baseline (speedup 1.0000x reference)
import functools

import jax
import jax.numpy as jnp
from jax import lax
from jax.experimental import pallas as pl
from jax.experimental.pallas import tpu as pltpu
from jax.experimental.pallas import tpu_sc as plsc

N = 10000
E = 160000
EH = E // 2
D = 256

NC = 2
NS = 16
NW = NC * NS
CHUNK = 64
NCHUNKS = EH // CHUNK
NSETS = 3
NP = 10240
ZROWS = NP // NS


def _sc_gather_body(half, hs_hbm, dst_hbm, src_hbm, gsum_hbm, cnt0_hbm, cnt1_hbm,
                    idx_d0, idx_d1, idx_d2, idx_s0, idx_s1, idx_s2,
                    buf_d0, buf_d1, buf_d2, buf_s0, buf_s1, buf_s2,
                    ones_v, zbuf, counts_sh,
                    gd0, gd1, gd2, gs0, gs1, gs2,
                    w0, w1, w2, ss0, ss1, ss2):
  idx_d = (idx_d0, idx_d1, idx_d2)
  idx_s = (idx_s0, idx_s1, idx_s2)
  buf_d = (buf_d0, buf_d1, buf_d2)
  buf_s = (buf_s0, buf_s1, buf_s2)
  gdsem = (gd0, gd1, gd2)
  gssem = (gs0, gs1, gs2)
  wsem = (w0, w1, w2)
  ssem = (ss0, ss1, ss2)

  c = lax.axis_index("c")
  s = lax.axis_index("s")
  wid = c * NS + s
  n = (NCHUNKS + NW - 1 - wid) // NW

  def fill_ones(i, _):
    ones_v[pl.ds(i * 16, 16)] = jnp.full((16,), 1.0, dtype=jnp.float32)
    return 0
  lax.fori_loop(0, CHUNK // 16, fill_ones, 0)

  def fill_zeros(i, _):
    zbuf[pl.ds(i * 16, 16)] = jnp.zeros((16,), dtype=jnp.float32)
    return 0
  lax.fori_loop(0, ZROWS // 16, fill_zeros, 0)

  pltpu.sync_copy(zbuf, counts_sh.at[pl.ds(s * ZROWS, ZROWS)])
  plsc.subcore_barrier()

  def cbase(k):
    return (half * NCHUNKS + wid + k * NW) * CHUNK

  def obase(k):
    return (wid + k * NW) * CHUNK

  def issue(k, q):
    pltpu.sync_copy(dst_hbm.at[pl.ds(cbase(k), CHUNK)], idx_d[q])
    pltpu.sync_copy(src_hbm.at[pl.ds(cbase(k), CHUNK)], idx_s[q])
    pltpu.async_copy(hs_hbm.at[idx_d[q]], buf_d[q], gdsem[q])
    pltpu.async_copy(hs_hbm.at[idx_s[q]], buf_s[q], gssem[q])

    @pl.when(k < n - NSETS)
    def _():
      pltpu.async_copy(ones_v, counts_sh.at[idx_d[q]], ssem[q], add=True)

    @pl.when(k >= n - NSETS)
    def _():
      pltpu.sync_copy(ones_v, counts_sh.at[idx_d[q]], add=True)

  def substep(k, p):
    q = (p + 1) % NSETS

    @pl.when(k < n)
    def _():
      @pl.when(k + 1 < n)
      def _():
        @pl.when(k >= 2)
        def _():
          pltpu.make_async_copy(
              buf_d[q], gsum_hbm.at[pl.ds(0, CHUNK)], wsem[q]).wait()
          pltpu.make_async_copy(
              cnt0_hbm.at[pl.ds(0, CHUNK)], ones_v, ssem[q]).wait()
        issue(k + 1, q)

      pltpu.make_async_copy(hs_hbm.at[idx_d[p]], buf_d[p], gdsem[p]).wait()
      pltpu.make_async_copy(hs_hbm.at[idx_s[p]], buf_s[p], gssem[p]).wait()

      def row_add(r, _):
        for j in range(D // 16):
          sl = pl.ds(j * 16, 16)
          buf_d[p][r, sl] = buf_d[p][r, sl] + buf_s[p][r, sl]
        return 0
      lax.fori_loop(0, CHUNK, row_add, 0)

      @pl.when(k < n - NSETS)
      def _():
        pltpu.async_copy(buf_d[p], gsum_hbm.at[pl.ds(obase(k), CHUNK)], wsem[p])

      @pl.when(k >= n - NSETS)
      def _():
        pltpu.sync_copy(buf_d[p], gsum_hbm.at[pl.ds(obase(k), CHUNK)])

  issue(0, 0)

  def ring_body(g, _):
    substep(3 * g, 0)
    substep(3 * g + 1, 1)
    substep(3 * g + 2, 2)
    return 0
  lax.fori_loop(0, (n + NSETS - 1) // NSETS, ring_body, 0)

  plsc.subcore_barrier()

  @pl.when((s == 0) & (c == 0))
  def _():
    pltpu.sync_copy(counts_sh, cnt0_hbm)

  @pl.when((s == 0) & (c == 1))
  def _():
    pltpu.sync_copy(counts_sh, cnt1_hbm)


@functools.partial(jax.jit, static_argnums=(3,))
def _sc_gather(hs, dst, src, half):
  mesh = plsc.VectorSubcoreMesh(core_axis_name="c", subcore_axis_name="s")
  return pl.kernel(
      functools.partial(_sc_gather_body, half),
      out_type=(
          jax.ShapeDtypeStruct((EH, D), jnp.float32),
          jax.ShapeDtypeStruct((NP,), jnp.float32),
          jax.ShapeDtypeStruct((NP,), jnp.float32),
      ),
      mesh=mesh,
      scratch_types=(
          pltpu.VMEM((CHUNK,), jnp.int32),
          pltpu.VMEM((CHUNK,), jnp.int32),
          pltpu.VMEM((CHUNK,), jnp.int32),
          pltpu.VMEM((CHUNK,), jnp.int32),
          pltpu.VMEM((CHUNK,), jnp.int32),
          pltpu.VMEM((CHUNK,), jnp.int32),
          pltpu.VMEM((CHUNK, D), jnp.float32),
          pltpu.VMEM((CHUNK, D), jnp.float32),
          pltpu.VMEM((CHUNK, D), jnp.float32),
          pltpu.VMEM((CHUNK, D), jnp.float32),
          pltpu.VMEM((CHUNK, D), jnp.float32),
          pltpu.VMEM((CHUNK, D), jnp.float32),
          pltpu.VMEM((CHUNK,), jnp.float32),
          pltpu.VMEM((ZROWS,), jnp.float32),
          pltpu.VMEM_SHARED((NP,), jnp.float32),
          pltpu.SemaphoreType.DMA,
          pltpu.SemaphoreType.DMA,
          pltpu.SemaphoreType.DMA,
          pltpu.SemaphoreType.DMA,
          pltpu.SemaphoreType.DMA,
          pltpu.SemaphoreType.DMA,
          pltpu.SemaphoreType.DMA,
          pltpu.SemaphoreType.DMA,
          pltpu.SemaphoreType.DMA,
          pltpu.SemaphoreType.DMA,
          pltpu.SemaphoreType.DMA,
          pltpu.SemaphoreType.DMA,
      ),
  )(hs, dst, src)


BE = 3200
GRIDE = EH // BE
BN = N // GRIDE


def _tc_edge_body(hs_e_ref, gsum_ref, wt_ref, b_ref, out_e_ref):
  he = hs_e_ref[...]
  y = jnp.dot(he.astype(jnp.bfloat16), wt_ref[...],
              preferred_element_type=jnp.float32)
  out_e_ref[...] = jax.nn.sigmoid(y + b_ref[...]) + gsum_ref[...] + he


def _tc_final_body(part_ref, hs_e_ref, gsum_ref, wt_ref, b_ref, hs_ref,
                   c0_ref, c1_ref, c2_ref, c3_ref, out_e_ref, out_n_ref):
  del part_ref
  he = hs_e_ref[...]
  y = jnp.dot(he.astype(jnp.bfloat16), wt_ref[...],
              preferred_element_type=jnp.float32)
  out_e_ref[...] = jax.nn.sigmoid(y + b_ref[...]) + gsum_ref[...] + he
  m = (c0_ref[...] + c1_ref[...] + c2_ref[...] + c3_ref[...]) > 0.0
  out_n_ref[...] = jnp.where(m, hs_ref[...], 0.0)


@jax.jit
def _tc_edge(hs_e, gsum, wt, b2):
  return pl.pallas_call(
      _tc_edge_body,
      grid=(GRIDE,),
      in_specs=[
          pl.BlockSpec((BE, D), lambda i: (i, 0)),
          pl.BlockSpec((BE, D), lambda i: (i, 0)),
          pl.BlockSpec((D, D), lambda i: (0, 0)),
          pl.BlockSpec((1, D), lambda i: (0, 0)),
      ],
      out_specs=pl.BlockSpec((BE, D), lambda i: (i, 0)),
      out_shape=jax.ShapeDtypeStruct((E, D), jnp.float32),
      compiler_params=pltpu.CompilerParams(
          dimension_semantics=("arbitrary",),
      ),
  )(hs_e, gsum, wt, b2)


@jax.jit
def _tc_final(part, hs_e, gsum, wt, b2, hs, c0, c1, c2, c3):
  return pl.pallas_call(
      _tc_final_body,
      grid=(GRIDE,),
      in_specs=[
          pl.BlockSpec(memory_space=pl.ANY),
          pl.BlockSpec((BE, D), lambda i: (i + GRIDE, 0)),
          pl.BlockSpec((BE, D), lambda i: (i, 0)),
          pl.BlockSpec((D, D), lambda i: (0, 0)),
          pl.BlockSpec((1, D), lambda i: (0, 0)),
          pl.BlockSpec((BN, D), lambda i: (i, 0)),
          pl.BlockSpec((BN, 1), lambda i: (i, 0)),
          pl.BlockSpec((BN, 1), lambda i: (i, 0)),
          pl.BlockSpec((BN, 1), lambda i: (i, 0)),
          pl.BlockSpec((BN, 1), lambda i: (i, 0)),
      ],
      out_specs=[
          pl.BlockSpec((BE, D), lambda i: (i + GRIDE, 0)),
          pl.BlockSpec((BN, D), lambda i: (i, 0)),
      ],
      out_shape=[
          jax.ShapeDtypeStruct((E, D), jnp.float32),
          jax.ShapeDtypeStruct((N, D), jnp.float32),
      ],
      input_output_aliases={0: 0},
      compiler_params=pltpu.CompilerParams(
          dimension_semantics=("arbitrary",),
      ),
  )(part, hs_e, gsum, wt, b2, hs, c0, c1, c2, c3)


def kernel(hs, hs_e, edge_index, W, b):
  src = edge_index[0]
  dst = edge_index[1]
  wt = W.T.astype(jnp.bfloat16)
  b2 = b.reshape(1, D)

  gsum0, c00, c01 = _sc_gather(hs, dst, src, 0)
  gsum1, c10, c11 = _sc_gather(hs, dst, src, 1)

  part = _tc_edge(hs_e, gsum0, wt, b2)
  new_hs_e, useless = _tc_final(
      part, hs_e, gsum1, wt, b2, hs,
      c00.reshape(NP, 1), c01.reshape(NP, 1),
      c10.reshape(NP, 1), c11.reshape(NP, 1))
  return new_hs_e, useless
